# baseline (speedup 1.0000x reference)
import jax
import jax.numpy as jnp
import numpy as np
from jax.experimental import pallas as pl

_PATCH = 16
_SIDELEN = 32
_T = 16


def _build_row_mask() -> np.ndarray:
    num_patches = _SIDELEN * _SIDELEN
    num_masked = num_patches // 2
    keys = jax.random.split(jax.random.key(42), _T)
    perms = jax.vmap(lambda k: jax.random.permutation(k, num_patches))(keys)
    idx = np.asarray(perms[:, :num_masked])
    h = idx % _SIDELEN
    w = idx // _SIDELEN
    pm = np.ones((_T, _SIDELEN, _SIDELEN), np.float32)
    pm[np.arange(_T)[:, None], h, w] = 0.0
    return np.repeat(pm, _PATCH, axis=2)


_ROW_MASK = _build_row_mask()


def _mask_body(f_ref, m_ref, o_ref):
    o_ref[0, 0] = f_ref[0, 0] * m_ref[0]


def kernel(frames):
    C, T, H, W = frames.shape
    s = H // _PATCH
    f5 = frames.reshape(C, T, s, _PATCH, W)
    mask = jnp.asarray(_ROW_MASK)[:, :, None, :]
    PB = 8
    grid = (T, s // PB, C)
    out5 = pl.pallas_call(
        _mask_body,
        grid=grid,
        in_specs=[
            pl.BlockSpec((1, 1, PB, _PATCH, W), lambda t, p, c: (c, t, p, 0, 0)),
            pl.BlockSpec((1, PB, 1, W), lambda t, p, c: (t, p, 0, 0)),
        ],
        out_specs=pl.BlockSpec((1, 1, PB, _PATCH, W), lambda t, p, c: (c, t, p, 0, 0)),
        out_shape=jax.ShapeDtypeStruct(f5.shape, f5.dtype),
    )(f5, mask)
    return out5.reshape(C, T, H, W)
